# Initial kernel scaffold; baseline (speedup 1.0000x reference)
import jax
import jax.numpy as jnp
from jax.experimental import pallas as pl


def kernel(edge_index, user_preference, features, W, b):
    raise NotImplementedError("write your pallas kernel here")


# speedup vs baseline: 1.0271x; 1.0271x over previous
import jax
import jax.numpy as jnp
from jax.experimental import pallas as pl
from jax.experimental.pallas import tpu as pltpu


def _mm_body(f_ref, wt_ref, b_ref, o_ref):
    o_ref[...] = jnp.dot(f_ref[...], wt_ref[...],
                         preferred_element_type=jnp.float32) + b_ref[...]


def _item_transform(features, W, b):
    N, F = features.shape
    E = W.shape[0]
    return pl.pallas_call(
        _mm_body,
        out_shape=jax.ShapeDtypeStruct((N, E), jnp.float32),
        grid=(N // 1000,),
        in_specs=[
            pl.BlockSpec((1000, F), lambda i: (i, 0)),
            pl.BlockSpec((F, E), lambda i: (0, 0)),
            pl.BlockSpec((1, E), lambda i: (0, 0)),
        ],
        out_specs=pl.BlockSpec((1000, E), lambda i: (i, 0)),
    )(features, W.T, b[None, :])


def _layer(x, edge_index):
    row = edge_index[0]
    col = edge_index[1]
    E = row.shape[0]
    deg = jnp.zeros((E,), dtype=x.dtype).at[row].add(1.0)
    dis = jnp.power(deg, -0.5)
    norm = dis[row] * dis[col]
    msg = norm[:, None] * x[row]
    return jnp.zeros_like(x).at[col].add(msg)


def kernel(edge_index, user_preference, features, W, b):
    item_features = _item_transform(features, W, b)
    x = jnp.concatenate([user_preference, item_features], axis=0)
    nrm = jnp.sqrt(jnp.sum(x * x, axis=1, keepdims=True))
    x = x / jnp.maximum(nrm, 1e-12)
    h1 = _layer(x, edge_index)
    h2 = _layer(h1, edge_index)
    return x + h1 + h2


# speedup vs baseline: 2.8256x; 2.7511x over previous
import functools

import jax
import jax.numpy as jnp
from jax import lax
from jax.experimental import pallas as pl
from jax.experimental.pallas import tpu as pltpu
from jax.experimental.pallas import tpu_sc as plsc

N_USERS = 5000
N_ITEMS = 5000
N_NODES = 10000
D = 256
DH = 128
E = 160000

NP = 10240
SLAB = NP // 16
CH = 128
NCH = 80
IBLK = 16
NIB = NCH // IBLK
EP16 = NCH * CH
EP = EP16 * 16

_mesh = plsc.VectorSubcoreMesh(core_axis_name="c", subcore_axis_name="s")


@functools.partial(
    pl.kernel,
    out_type=jax.ShapeDtypeStruct((NP, DH), jnp.float32),
    mesh=_mesh,
    scratch_types=[
        pltpu.VMEM((NCH, CH), jnp.int32),
        pltpu.VMEM((CH, DH), jnp.float32),
        pltpu.VMEM_SHARED((NP, DH), jnp.float32),
    ],
)
def _deg_kernel(rows_hbm, ones_hbm, zeros8_hbm, deg_hbm, ridx, onesv, hist):
    c = lax.axis_index("c")
    s = lax.axis_index("s")
    slab = pl.ds(s * SLAB, SLAB)
    pltpu.sync_copy(rows_hbm.at[0, s], ridx)
    pltpu.sync_copy(ones_hbm, onesv)
    pltpu.sync_copy(zeros8_hbm.at[slab], hist.at[slab])
    plsc.subcore_barrier()

    def body(j, carry):
        pltpu.sync_copy(onesv, hist.at[ridx.at[j]], add=True)
        return carry

    lax.fori_loop(0, NCH, body, 0)
    plsc.subcore_barrier()

    @pl.when(c == 0)
    def _():
        pltpu.sync_copy(hist.at[slab], deg_hbm.at[slab])


def _mm_body(f_ref, wt_ref, b_ref, o_ref):
    o_ref[...] = jnp.dot(f_ref[...], wt_ref[...],
                         preferred_element_type=jnp.float32) + b_ref[...]


def _item_transform(features, W, b):
    return pl.pallas_call(
        _mm_body,
        out_shape=jax.ShapeDtypeStruct((N_ITEMS, D), jnp.float32),
        grid=(N_ITEMS // 1000,),
        in_specs=[
            pl.BlockSpec((1000, D), lambda i: (i, 0)),
            pl.BlockSpec((D, D), lambda i: (0, 0)),
            pl.BlockSpec((1, D), lambda i: (0, 0)),
        ],
        out_specs=pl.BlockSpec((1000, D), lambda i: (i, 0)),
    )(features, W.T, b[None, :])


def _norm_body(x_ref, dg_ref, xs_ref, ys_ref, dis_ref):
    x = x_ref[...]
    deg = dg_ref[...]
    dis = jnp.where(deg > 0, lax.rsqrt(deg), 0.0)
    nrm = jnp.sqrt(jnp.sum(x * x, axis=1, keepdims=True))
    xn = x / jnp.maximum(nrm, 1e-12)
    y = dis * xn
    xs_ref[0] = xn[:, :DH]
    xs_ref[1] = xn[:, DH:]
    ys_ref[0] = y[:, :DH]
    ys_ref[1] = y[:, DH:]
    dis_ref[...] = dis


def _normalize_split(xraw_pad, deg_col):
    blk = 256
    return pl.pallas_call(
        _norm_body,
        out_shape=(
            jax.ShapeDtypeStruct((2, NP, DH), jnp.float32),
            jax.ShapeDtypeStruct((2, NP, DH), jnp.float32),
            jax.ShapeDtypeStruct((NP, 1), jnp.float32),
        ),
        grid=(NP // blk,),
        in_specs=[
            pl.BlockSpec((blk, D), lambda i: (i, 0)),
            pl.BlockSpec((blk, 1), lambda i: (i, 0)),
        ],
        out_specs=(
            pl.BlockSpec((2, blk, DH), lambda i: (0, i, 0)),
            pl.BlockSpec((2, blk, DH), lambda i: (0, i, 0)),
            pl.BlockSpec((blk, 1), lambda i: (i, 0)),
        ),
    )(xraw_pad, deg_col)


@functools.partial(
    pl.kernel,
    out_type=jax.ShapeDtypeStruct((2, NP, DH), jnp.float32),
    mesh=_mesh,
    scratch_types=[
        pltpu.VMEM((IBLK, CH), jnp.int32),
        pltpu.VMEM((IBLK, CH), jnp.int32),
        pltpu.VMEM((CH, DH), jnp.float32),
        pltpu.VMEM_SHARED((NP, DH), jnp.float32),
        pltpu.SemaphoreType.DMA,
    ],
)
def _agg_kernel(rows_hbm, cols_hbm, ytab_hbm, zeros_hbm, acc_hbm,
                ridx, cidx, gbuf, acc, sem):
    c = lax.axis_index("c")
    s = lax.axis_index("s")
    slab = pl.ds(s * SLAB, SLAB)
    pltpu.sync_copy(zeros_hbm.at[slab], acc.at[slab])
    plsc.subcore_barrier()

    def outer(ib, carry):
        isl = pl.ds(ib * IBLK, IBLK)
        pltpu.sync_copy(rows_hbm.at[c, s, isl], ridx)
        pltpu.sync_copy(cols_hbm.at[s, isl], cidx)

        def inner(j, carry2):
            pltpu.async_copy(ytab_hbm.at[ridx.at[j]], gbuf, sem).wait()
            pltpu.sync_copy(gbuf, acc.at[cidx.at[j]], add=True)
            return carry2

        lax.fori_loop(0, IBLK, inner, 0)
        return carry

    lax.fori_loop(0, NIB, outer, 0)
    plsc.subcore_barrier()
    pltpu.sync_copy(acc.at[slab], acc_hbm.at[c, slab])


def _mid_body(a_ref, dis_ref, h1_ref, y2_ref):
    a = a_ref[0]
    dis = dis_ref[...]
    h1 = dis * a
    h1_ref[0] = h1
    y2_ref[0] = dis * h1


def _mid_scale(acc1, disv):
    blk = 512
    return pl.pallas_call(
        _mid_body,
        out_shape=(
            jax.ShapeDtypeStruct((2, NP, DH), jnp.float32),
            jax.ShapeDtypeStruct((2, NP, DH), jnp.float32),
        ),
        grid=(2, NP // blk),
        in_specs=[
            pl.BlockSpec((1, blk, DH), lambda c, i: (c, i, 0)),
            pl.BlockSpec((blk, 1), lambda c, i: (i, 0)),
        ],
        out_specs=(
            pl.BlockSpec((1, blk, DH), lambda c, i: (c, i, 0)),
            pl.BlockSpec((1, blk, DH), lambda c, i: (c, i, 0)),
        ),
    )(acc1, disv)


def _fin_body(x_ref, h1_ref, a_ref, dis_ref, o_ref):
    o_ref[0] = x_ref[0] + h1_ref[0] + dis_ref[...] * a_ref[0]


def _final_sum(xs, h1s, acc2, disv):
    blk = 512
    return pl.pallas_call(
        _fin_body,
        out_shape=jax.ShapeDtypeStruct((2, NP, DH), jnp.float32),
        grid=(2, NP // blk),
        in_specs=[
            pl.BlockSpec((1, blk, DH), lambda c, i: (c, i, 0)),
            pl.BlockSpec((1, blk, DH), lambda c, i: (c, i, 0)),
            pl.BlockSpec((1, blk, DH), lambda c, i: (c, i, 0)),
            pl.BlockSpec((blk, 1), lambda c, i: (i, 0)),
        ],
        out_specs=pl.BlockSpec((1, blk, DH), lambda c, i: (c, i, 0)),
    )(xs, h1s, acc2, disv)


def kernel(edge_index, user_preference, features, W, b):
    i32 = jnp.int32
    row = edge_index[0].astype(i32)
    col = edge_index[1].astype(i32)
    pad = jnp.full((EP - E,), N_NODES, dtype=i32)
    rows16 = jnp.concatenate([row, pad]).reshape(16, NCH, CH)
    cols16 = jnp.concatenate([col, pad]).reshape(16, NCH, CH)
    rowg = jnp.stack([rows16, rows16 + NP])

    ones_big = jnp.ones((CH, DH), jnp.float32)
    zeros_big = jnp.zeros((NP, DH), jnp.float32)

    deg8 = _deg_kernel(rowg, ones_big, zeros_big)
    deg_col = deg8[:, :1]

    item_features = _item_transform(features, W, b)
    xraw = jnp.concatenate([user_preference, item_features], axis=0)
    xraw_pad = jnp.pad(xraw, ((0, NP - N_NODES), (0, 0)))

    xs3, ys3, disv = _normalize_split(xraw_pad, deg_col)
    ys = ys3.reshape(2 * NP, DH)

    acc1 = _agg_kernel(rowg, cols16, ys, zeros_big)
    h1s, y2s = _mid_scale(acc1, disv)
    acc2 = _agg_kernel(rowg, cols16, y2s.reshape(2 * NP, DH), zeros_big)
    outs = _final_sum(xs3, h1s, acc2, disv)

    out = outs.transpose(1, 0, 2)[:N_NODES]
    return out.reshape(N_NODES, D)


# speedup vs baseline: 3.2585x; 1.1532x over previous
import functools

import jax
import jax.numpy as jnp
from jax import lax
from jax.experimental import pallas as pl
from jax.experimental.pallas import tpu as pltpu
from jax.experimental.pallas import tpu_sc as plsc

N_USERS = 5000
N_ITEMS = 5000
N_NODES = 10000
D = 256
DH = 128
E = 160000

NP = 10240
SLAB = NP // 16
CH = 128
NCH = 80
IBLK = 16
NIB = NCH // IBLK
EP16 = NCH * CH
EP = EP16 * 16

_mesh = plsc.VectorSubcoreMesh(core_axis_name="c", subcore_axis_name="s")


@functools.partial(
    pl.kernel,
    out_type=jax.ShapeDtypeStruct((NP, DH), jnp.float32),
    mesh=_mesh,
    scratch_types=[
        pltpu.VMEM((NCH, CH), jnp.int32),
        pltpu.VMEM((CH, DH), jnp.float32),
        pltpu.VMEM_SHARED((NP, DH), jnp.float32),
        pltpu.SemaphoreType.DMA,
    ],
)
def _deg_kernel(rows_hbm, ones_hbm, zeros8_hbm, deg_hbm, ridx, onesv, hist,
                sem):
    c = lax.axis_index("c")
    s = lax.axis_index("s")
    slab = pl.ds(s * SLAB, SLAB)
    pltpu.sync_copy(rows_hbm.at[0, s], ridx)
    pltpu.sync_copy(ones_hbm, onesv)
    pltpu.sync_copy(zeros8_hbm.at[slab], hist.at[slab])
    plsc.subcore_barrier()

    def body(g, carry):
        for t in range(8):
            pltpu.async_copy(onesv, hist.at[ridx.at[g * 8 + t]], sem,
                             add=True)
        for t in range(8):
            pltpu.make_async_copy(onesv, hist.at[ridx.at[g * 8]], sem).wait()
        return carry

    lax.fori_loop(0, NCH // 8, body, 0)
    plsc.subcore_barrier()

    @pl.when(c == 0)
    def _():
        pltpu.sync_copy(hist.at[slab], deg_hbm.at[slab])


def _mm_body(f_ref, wt_ref, b_ref, o_ref):
    o_ref[...] = jnp.dot(f_ref[...], wt_ref[...],
                         preferred_element_type=jnp.float32) + b_ref[...]


def _item_transform(features, W, b):
    return pl.pallas_call(
        _mm_body,
        out_shape=jax.ShapeDtypeStruct((N_ITEMS, D), jnp.float32),
        grid=(N_ITEMS // 1000,),
        in_specs=[
            pl.BlockSpec((1000, D), lambda i: (i, 0)),
            pl.BlockSpec((D, D), lambda i: (0, 0)),
            pl.BlockSpec((1, D), lambda i: (0, 0)),
        ],
        out_specs=pl.BlockSpec((1000, D), lambda i: (i, 0)),
    )(features, W.T, b[None, :])


def _norm_body(x_ref, dg_ref, xs_ref, ys_ref, dis_ref):
    x = x_ref[...]
    deg = dg_ref[...]
    dis = jnp.where(deg > 0, lax.rsqrt(deg), 0.0)
    nrm = jnp.sqrt(jnp.sum(x * x, axis=1, keepdims=True))
    xn = x / jnp.maximum(nrm, 1e-12)
    y = dis * xn
    xs_ref[0] = xn[:, :DH]
    xs_ref[1] = xn[:, DH:]
    ys_ref[0] = y[:, :DH]
    ys_ref[1] = y[:, DH:]
    dis_ref[...] = dis


def _normalize_split(xraw_pad, deg_col):
    blk = 256
    return pl.pallas_call(
        _norm_body,
        out_shape=(
            jax.ShapeDtypeStruct((2, NP, DH), jnp.float32),
            jax.ShapeDtypeStruct((2, NP, DH), jnp.float32),
            jax.ShapeDtypeStruct((NP, 1), jnp.float32),
        ),
        grid=(NP // blk,),
        in_specs=[
            pl.BlockSpec((blk, D), lambda i: (i, 0)),
            pl.BlockSpec((blk, 1), lambda i: (i, 0)),
        ],
        out_specs=(
            pl.BlockSpec((2, blk, DH), lambda i: (0, i, 0)),
            pl.BlockSpec((2, blk, DH), lambda i: (0, i, 0)),
            pl.BlockSpec((blk, 1), lambda i: (i, 0)),
        ),
    )(xraw_pad, deg_col)


@functools.partial(
    pl.kernel,
    out_type=jax.ShapeDtypeStruct((2, NP, DH), jnp.float32),
    mesh=_mesh,
    scratch_types=[
        pltpu.VMEM((IBLK, CH), jnp.int32),
        pltpu.VMEM((IBLK, CH), jnp.int32),
        pltpu.VMEM((CH, DH), jnp.float32),
        pltpu.VMEM((CH, DH), jnp.float32),
        pltpu.VMEM_SHARED((NP, DH), jnp.float32),
        pltpu.SemaphoreType.DMA,
        pltpu.SemaphoreType.DMA,
    ],
)
def _agg_kernel(rows_hbm, cols_hbm, ytab_hbm, zeros_hbm, acc_hbm,
                ridx, cidx, ga, gb, acc, sema, semb):
    c = lax.axis_index("c")
    s = lax.axis_index("s")
    slab = pl.ds(s * SLAB, SLAB)
    pltpu.sync_copy(zeros_hbm.at[slab], acc.at[slab])
    plsc.subcore_barrier()

    def outer(ib, carry):
        isl = pl.ds(ib * IBLK, IBLK)
        pltpu.sync_copy(rows_hbm.at[c, s, isl], ridx)
        pltpu.sync_copy(cols_hbm.at[s, isl], cidx)
        pltpu.async_copy(ytab_hbm.at[ridx.at[0]], ga, sema)
        pltpu.async_copy(ytab_hbm.at[ridx.at[1]], gb, semb)

        def pair(p, carry2):
            j = 2 * p
            last = p >= IBLK // 2 - 1
            pltpu.make_async_copy(ytab_hbm.at[ridx.at[j]], ga, sema).wait()
            pltpu.sync_copy(ga, acc.at[cidx.at[j]], add=True)

            @pl.when(jnp.logical_not(last))
            def _():
                pltpu.async_copy(ytab_hbm.at[ridx.at[j + 2]], ga, sema)

            pltpu.make_async_copy(ytab_hbm.at[ridx.at[j + 1]], gb, semb).wait()
            pltpu.sync_copy(gb, acc.at[cidx.at[j + 1]], add=True)

            @pl.when(jnp.logical_not(last))
            def _():
                pltpu.async_copy(ytab_hbm.at[ridx.at[j + 3]], gb, semb)

            return carry2

        lax.fori_loop(0, IBLK // 2, pair, 0)
        return carry

    lax.fori_loop(0, NIB, outer, 0)
    plsc.subcore_barrier()
    pltpu.sync_copy(acc.at[slab], acc_hbm.at[c, slab])


def _mid_body(a_ref, dis_ref, h1_ref, y2_ref):
    a = a_ref[0]
    dis = dis_ref[...]
    h1 = dis * a
    h1_ref[0] = h1
    y2_ref[0] = dis * h1


def _mid_scale(acc1, disv):
    blk = 512
    return pl.pallas_call(
        _mid_body,
        out_shape=(
            jax.ShapeDtypeStruct((2, NP, DH), jnp.float32),
            jax.ShapeDtypeStruct((2, NP, DH), jnp.float32),
        ),
        grid=(2, NP // blk),
        in_specs=[
            pl.BlockSpec((1, blk, DH), lambda c, i: (c, i, 0)),
            pl.BlockSpec((blk, 1), lambda c, i: (i, 0)),
        ],
        out_specs=(
            pl.BlockSpec((1, blk, DH), lambda c, i: (c, i, 0)),
            pl.BlockSpec((1, blk, DH), lambda c, i: (c, i, 0)),
        ),
    )(acc1, disv)


def _fin_body(x_ref, h1_ref, a_ref, dis_ref, o_ref):
    o_ref[0] = x_ref[0] + h1_ref[0] + dis_ref[...] * a_ref[0]


def _final_sum(xs, h1s, acc2, disv):
    blk = 512
    return pl.pallas_call(
        _fin_body,
        out_shape=jax.ShapeDtypeStruct((2, NP, DH), jnp.float32),
        grid=(2, NP // blk),
        in_specs=[
            pl.BlockSpec((1, blk, DH), lambda c, i: (c, i, 0)),
            pl.BlockSpec((1, blk, DH), lambda c, i: (c, i, 0)),
            pl.BlockSpec((1, blk, DH), lambda c, i: (c, i, 0)),
            pl.BlockSpec((blk, 1), lambda c, i: (i, 0)),
        ],
        out_specs=pl.BlockSpec((1, blk, DH), lambda c, i: (c, i, 0)),
    )(xs, h1s, acc2, disv)


def kernel(edge_index, user_preference, features, W, b):
    i32 = jnp.int32
    row = edge_index[0].astype(i32)
    col = edge_index[1].astype(i32)
    pad = jnp.full((EP - E,), N_NODES, dtype=i32)
    rows16 = jnp.concatenate([row, pad]).reshape(16, NCH, CH)
    cols16 = jnp.concatenate([col, pad]).reshape(16, NCH, CH)
    rowg = jnp.stack([rows16, rows16 + NP])

    ones_big = jnp.ones((CH, DH), jnp.float32)
    zeros_big = jnp.zeros((NP, DH), jnp.float32)

    deg8 = _deg_kernel(rowg, ones_big, zeros_big)
    deg_col = deg8[:, :1]

    item_features = _item_transform(features, W, b)
    xraw = jnp.concatenate([user_preference, item_features], axis=0)
    xraw_pad = jnp.pad(xraw, ((0, NP - N_NODES), (0, 0)))

    xs3, ys3, disv = _normalize_split(xraw_pad, deg_col)
    ys = ys3.reshape(2 * NP, DH)

    acc1 = _agg_kernel(rowg, cols16, ys, zeros_big)
    h1s, y2s = _mid_scale(acc1, disv)
    acc2 = _agg_kernel(rowg, cols16, y2s.reshape(2 * NP, DH), zeros_big)
    outs = _final_sum(xs3, h1s, acc2, disv)

    out = outs.transpose(1, 0, 2)[:N_NODES]
    return out.reshape(N_NODES, D)
